# baseline (speedup 1.0000x reference)
import jax
import jax.numpy as jnp
from jax.experimental import pallas as pl
from jax.experimental.pallas import tpu as pltpu

_CHUNKS = (4992, 3328, 1536, 144)
_N = 10000
_MAXC = max(_CHUNKS)


def _fused_kernel(x_hbm, pk_ref, out_ref, buf, sems):
    offs = [sum(_CHUNKS[:i]) for i in range(len(_CHUNKS))]
    copies = [
        pltpu.make_async_copy(
            x_hbm.at[pl.ds(offs[ci], c), :], buf.at[ci, pl.ds(0, c)],
            sems.at[ci])
        for ci, c in enumerate(_CHUNKS)
    ]
    for c in copies:
        c.start()
    wcat = pk_ref[0:128, :].astype(jnp.bfloat16)
    bcat = pk_ref[128:192, 0:1]
    wlin = pk_ref[192:224, 0:1]
    blin = pk_ref[224:225, 0:1]
    hid = wlin.shape[0]
    for ci, c in enumerate(_CHUNKS):
        copies[ci].wait()
        x = buf[ci, 0:c].astype(jnp.bfloat16)
        lg = jax.lax.dot_general(wcat, x, (((0,), (1,)), ((), ())),
                                 preferred_element_type=jnp.float32) + bcat
        tau = jnp.tanh(lg)
        s1 = 1.0 + tau[:hid]
        t = tau[hid:]
        h = jax.nn.relu(s1 * t)
        o = jnp.sum(h * wlin, axis=0, keepdims=True)
        out_ref[0, 0:1, pl.ds(offs[ci], c)] = o + blin


def kernel(x, edge_index, edge_weight, W_xz, b_xz, W_hz, b_hz, W_xr, b_xr,
           W_hr, b_hr, W_xh, b_xh, W_hh, b_hh, W_lin, b_lin):
    n, f_in = x.shape
    hid = W_xz.shape[1]
    wcat = jnp.concatenate([-0.5 * W_xz, W_xh], axis=1)
    tail = jnp.concatenate([
        -0.5 * (b_xz + b_hz), b_xh + b_hh,
        0.5 * W_lin[:, 0],
        b_lin,
    ]).reshape(2 * hid + hid + 1, 1)
    pk = jnp.concatenate(
        [wcat, jnp.pad(tail, ((0, 0), (0, 2 * hid - 1)))], axis=0)

    vm = pl.BlockSpec(memory_space=pltpu.MemorySpace.VMEM)
    out_row = pl.pallas_call(
        _fused_kernel,
        in_specs=[
            pl.BlockSpec(memory_space=pltpu.MemorySpace.HBM),
            vm,
        ],
        out_specs=vm,
        out_shape=jax.ShapeDtypeStruct((1, 1, n), x.dtype),
        scratch_shapes=[
            pltpu.MemorySpace.VMEM((len(_CHUNKS), _MAXC, f_in), jnp.float32),
            pltpu.SemaphoreType.DMA((len(_CHUNKS),)),
        ],
    )(x, pk)
    return out_row.reshape(n, 1)


# speedup vs baseline: 1.0223x; 1.0223x over previous
import jax
import jax.numpy as jnp
from jax.experimental import pallas as pl
from jax.experimental.pallas import tpu as pltpu

_CHUNKS = (4992, 3328, 1680)
_N = 10000
_MAXC = max(_CHUNKS)


def _fused_kernel(x_hbm, pk_ref, out_ref, buf, sems):
    offs = [sum(_CHUNKS[:i]) for i in range(len(_CHUNKS))]
    copies = [
        pltpu.make_async_copy(
            x_hbm.at[pl.ds(offs[ci], c), :], buf.at[ci, pl.ds(0, c)],
            sems.at[ci])
        for ci, c in enumerate(_CHUNKS)
    ]
    for c in copies:
        c.start()
    wcat = pk_ref[0:128, :].astype(jnp.bfloat16)
    bcat = pk_ref[128:192, 0:1]
    wlin = pk_ref[192:224, 0:1]
    blin = pk_ref[224:225, 0:1]
    hid = wlin.shape[0]
    for ci, c in enumerate(_CHUNKS):
        copies[ci].wait()
        x = buf[ci, 0:c].astype(jnp.bfloat16)
        lg = jax.lax.dot_general(wcat, x, (((0,), (1,)), ((), ())),
                                 preferred_element_type=jnp.float32) + bcat
        tau = jnp.tanh(lg)
        s1 = 1.0 + tau[:hid]
        t = tau[hid:]
        h = jax.nn.relu(s1 * t)
        o = jnp.sum(h * wlin, axis=0, keepdims=True)
        out_ref[0, 0:1, pl.ds(offs[ci], c)] = o + blin


def kernel(x, edge_index, edge_weight, W_xz, b_xz, W_hz, b_hz, W_xr, b_xr,
           W_hr, b_hr, W_xh, b_xh, W_hh, b_hh, W_lin, b_lin):
    n, f_in = x.shape
    hid = W_xz.shape[1]
    wcat = jnp.concatenate([-0.5 * W_xz, W_xh], axis=1)
    tail = jnp.concatenate([
        -0.5 * (b_xz + b_hz), b_xh + b_hh,
        0.5 * W_lin[:, 0],
        b_lin,
    ]).reshape(2 * hid + hid + 1, 1)
    pk = jnp.concatenate(
        [wcat, jnp.pad(tail, ((0, 0), (0, 2 * hid - 1)))], axis=0)

    vm = pl.BlockSpec(memory_space=pltpu.MemorySpace.VMEM)
    out_row = pl.pallas_call(
        _fused_kernel,
        in_specs=[
            pl.BlockSpec(memory_space=pltpu.MemorySpace.HBM),
            vm,
        ],
        out_specs=vm,
        out_shape=jax.ShapeDtypeStruct((1, 1, n), x.dtype),
        scratch_shapes=[
            pltpu.MemorySpace.VMEM((len(_CHUNKS), _MAXC, f_in), jnp.float32),
            pltpu.SemaphoreType.DMA((len(_CHUNKS),)),
        ],
    )(x, pk)
    return out_row.reshape(n, 1)


# speedup vs baseline: 1.0566x; 1.0335x over previous
import jax
import jax.numpy as jnp
from jax.experimental import pallas as pl
from jax.experimental.pallas import tpu as pltpu

_CHUNKS = (4992, 3328, 1680)
_N = 10000
_MAXC = max(_CHUNKS)


def _fused_kernel(x_hbm, pk_hbm, out_ref, buf, pk_b, sems, csem):
    offs = [sum(_CHUNKS[:i]) for i in range(len(_CHUNKS))]
    pk_copy = pltpu.make_async_copy(pk_hbm, pk_b, csem)
    pk_copy.start()
    copies = [
        pltpu.make_async_copy(
            x_hbm.at[pl.ds(offs[ci], c), :], buf.at[ci, pl.ds(0, c)],
            sems.at[ci])
        for ci, c in enumerate(_CHUNKS)
    ]
    for c in copies:
        c.start()
    pk_copy.wait()
    wcat = pk_b[0:128, :].astype(jnp.bfloat16)
    bcat = pk_b[128:192, 0:1]
    wlin = pk_b[192:224, 0:1]
    blin = pk_b[224:225, 0:1]
    hid = wlin.shape[0]
    for ci, c in enumerate(_CHUNKS):
        copies[ci].wait()
        x = buf[ci, 0:c].astype(jnp.bfloat16)
        lg = jax.lax.dot_general(wcat, x, (((0,), (1,)), ((), ())),
                                 preferred_element_type=jnp.float32) + bcat
        tau = jnp.tanh(lg)
        s1 = 1.0 + tau[:hid]
        t = tau[hid:]
        h = jax.nn.relu(s1 * t)
        o = jnp.sum(h * wlin, axis=0, keepdims=True)
        out_ref[0, 0:1, pl.ds(offs[ci], c)] = o + blin


def kernel(x, edge_index, edge_weight, W_xz, b_xz, W_hz, b_hz, W_xr, b_xr,
           W_hr, b_hr, W_xh, b_xh, W_hh, b_hh, W_lin, b_lin):
    n, f_in = x.shape
    hid = W_xz.shape[1]
    wcat = jnp.concatenate([-0.5 * W_xz, W_xh], axis=1)
    tail = jnp.concatenate([
        -0.5 * (b_xz + b_hz), b_xh + b_hh,
        0.5 * W_lin[:, 0],
        b_lin,
    ]).reshape(2 * hid + hid + 1, 1)
    pk = jnp.concatenate(
        [wcat, jnp.pad(tail, ((0, 0), (0, 2 * hid - 1)))], axis=0)

    vm = pl.BlockSpec(memory_space=pltpu.MemorySpace.VMEM)
    out_row = pl.pallas_call(
        _fused_kernel,
        in_specs=[
            pl.BlockSpec(memory_space=pltpu.MemorySpace.HBM),
            pl.BlockSpec(memory_space=pltpu.MemorySpace.HBM),
        ],
        out_specs=vm,
        out_shape=jax.ShapeDtypeStruct((1, 1, n), x.dtype),
        scratch_shapes=[
            pltpu.MemorySpace.VMEM((len(_CHUNKS), _MAXC, f_in), jnp.float32),
            pltpu.MemorySpace.VMEM((225, 2 * hid), jnp.float32),
            pltpu.SemaphoreType.DMA((len(_CHUNKS),)),
            pltpu.SemaphoreType.DMA,
        ],
    )(x, pk)
    return out_row.reshape(n, 1)
